# baseline (speedup 1.0000x reference)
import jax
import jax.numpy as jnp
from jax import lax
from jax.experimental import pallas as pl
from jax.experimental.pallas import tpu as pltpu
from jax.experimental.pallas import tpu_sc as plsc

GRID = 64
G3 = GRID * GRID * GRID
N = 100000
NTILES = 16
PPT = 6272
NPAD = NTILES * PPT
ROWS = PPT // 16
STRIPE = G3 // NTILES
NQ = 8
QB = STRIPE // NQ
THRESH = 0.7569


def _splat(v, dt=jnp.float32):
    return lax.broadcast(jnp.asarray(v, dt), (16,))


def _body(x_hbm, out_hbm, xin0, xin1, xin2, idxb, v0, v1, v2,
          idxc, w0, w1, w2, ones, zbuf,
          na0, na1, na2, na3, no0, no1, no2,
          nb0, nb1, nb2, nb3, np0, np1, np2,
          sema, semb, semz, semx, semn0, semn1, semo0, semo1,
          acc0, acc1, acc2, acc3):
    core = lax.axis_index("c")
    sid = lax.axis_index("s")
    accs = (acc0, acc1, acc2, acc3)
    xins = (xin0, xin1, xin2)
    nas = ((na0, na1, na2, na3), (nb0, nb1, nb2, nb3))
    nos = ((no0, no1, no2), (np0, np1, np2))
    semn = (semn0, semn1)
    semo = (semo0, semo1)

    def zrow(i, _):
        zbuf[pl.ds(i * 16, 16)] = _splat(0.0)
        return 0
    lax.fori_loop(0, QB // 16, zrow, 0)
    for k in range(8):
        ones[pl.ds(k * 16, 16)] = _splat(1.0)
    dumpv = lax.broadcast(G3 + sid, (16,))

    for rep in range(2):
        b = core * 2 + rep

        base = sid * PPT
        for ax in range(3):
            pltpu.async_copy(
                x_hbm.at[pl.ds((b * 3 + ax) * NPAD + base, PPT)], xins[ax],
                semx)

        def zfire(q):
            for accr in accs:
                pltpu.async_copy(zbuf, accr.at[pl.ds(sid * STRIPE + q * QB, QB)],
                                 semz)

        def zwait(q):
            for accr in accs:
                pltpu.make_async_copy(
                    zbuf, accr.at[pl.ds(sid * STRIPE + q * QB, QB)],
                    semz).wait()

        zfire(0)

        def zplane(q, _):
            zfire(q)
            zwait(q - 1)
            return 0
        lax.fori_loop(1, NQ, zplane, 0)
        zwait(NQ - 1)
        for ax in range(3):
            pltpu.make_async_copy(
                x_hbm.at[pl.ds((b * 3 + ax) * NPAD + base, PPT)], xins[ax],
                semx).wait()
        plsc.subcore_barrier()

        def compute_row(j, idxr, vr):
            p = j * 16
            f32 = jnp.float32
            i32 = jnp.int32
            half = _splat(0.5)
            g = _splat(64.0)
            zi = _splat(0, i32)
            mi = _splat(GRID - 1, i32)
            onei = _splat(1, i32)
            i1 = []
            i2 = []
            r1 = []
            r2 = []
            s1 = []
            s2 = []
            for ax in range(3):
                ta = (xins[ax][pl.ds(p, 16)] + half) * g - half
                ia = ta.astype(i32)
                up = jnp.where(ta > ia.astype(f32), onei, zi)
                ib = ia + up
                ia = jnp.minimum(jnp.maximum(ia, zi), mi)
                ib = jnp.minimum(jnp.maximum(ib, zi), mi)
                ra = ta - ia.astype(f32)
                rb = ta - ib.astype(f32)
                i1.append(ia)
                i2.append(ib)
                r1.append(ra)
                r2.append(rb)
                s1.append(ra * ra)
                s2.append(rb * rb)
            s12 = _splat(12, i32)
            s6 = _splat(6, i32)
            X = (lax.shift_left(i1[0], s12), lax.shift_left(i2[0], s12))
            Y = (lax.shift_left(i1[1], s6), lax.shift_left(i2[1], s6))
            Z = (i1[2], i2[2])
            SX = (s1[0], s2[0])
            SY = (s1[1], s2[1])
            SZ = (s1[2], s2[2])
            RX = (r1[0], r2[0])
            RY = (r1[1], r2[1])
            RZ = (r1[2], r2[2])
            th = _splat(THRESH)
            c = 0
            for a in range(2):
                for bb in range(2):
                    for cc in range(2):
                        d2 = (SX[a] + SY[bb]) + SZ[cc]
                        idx = (X[a] + Y[bb]) + Z[cc]
                        idx = jnp.where(d2 < th, idx, dumpv)
                        sl = pl.ds(c * 16, 16)
                        idxr[sl] = idx
                        vr[0][sl] = RX[a]
                        vr[1][sl] = RY[bb]
                        vr[2][sl] = RZ[cc]
                        c += 1

        def fire(idxr, vr, sem):
            for accr, v in zip(accs, vr + (ones,)):
                pltpu.async_copy(v, accr.at[idxr], sem, add=True)

        def drain(idxr, vr, sem):
            for accr, v in zip(accs, vr + (ones,)):
                pltpu.make_async_copy(v, accr.at[idxr], sem).wait()

        vsa = (v0, v1, v2)
        vsb = (w0, w1, w2)
        compute_row(0, idxb, vsa)
        fire(idxb, vsa, sema)
        compute_row(1, idxc, vsb)
        fire(idxc, vsb, semb)

        def pair(m, _):
            drain(idxb, vsa, sema)
            compute_row(2 * m, idxb, vsa)
            fire(idxb, vsa, sema)
            drain(idxc, vsb, semb)
            compute_row(2 * m + 1, idxc, vsb)
            fire(idxc, vsb, semb)
            return 0

        lax.fori_loop(1, ROWS // 2, pair, 0)
        drain(idxb, vsa, sema)
        drain(idxc, vsb, semb)
        plsc.subcore_barrier()

        def nfire_in(q, s):
            nb = sid * STRIPE + q * QB
            for ch in range(4):
                pltpu.async_copy(accs[ch].at[pl.ds(nb, QB)], nas[s][ch],
                                 semn[s])

        def nwait_in(q, s):
            nb = sid * STRIPE + q * QB
            for ch in range(4):
                pltpu.make_async_copy(accs[ch].at[pl.ds(nb, QB)], nas[s][ch],
                                      semn[s]).wait()

        def nfire_out(q, s):
            nb = sid * STRIPE + q * QB
            for ch in range(3):
                pltpu.async_copy(
                    nos[s][ch], out_hbm.at[pl.ds((b * 3 + ch) * G3 + nb, QB)],
                    semo[s])

        def nwait_out(q, s):
            nb = sid * STRIPE + q * QB
            for ch in range(3):
                pltpu.make_async_copy(
                    nos[s][ch], out_hbm.at[pl.ds((b * 3 + ch) * G3 + nb, QB)],
                    semo[s]).wait()

        def ncompute(s):
            a0, a1, a2, a3 = nas[s]
            o0, o1, o2 = nos[s]

            def nrow(i, _):
                sl = pl.ds(i * 16, 16)
                w = jnp.maximum(a3[sl], _splat(1.0))
                o0[sl] = a0[sl] / w
                o1[sl] = a1[sl] / w
                o2[sl] = a2[sl] / w
                return 0

            lax.fori_loop(0, QB // 16, nrow, 0)

        nfire_in(0, 0)

        def nchunk(h, _):
            for s in range(2):
                q = 2 * h + s
                nwait_in(q, s)

                @pl.when(q + 1 < NQ)
                def _():
                    nfire_in(q + 1, 1 - s)

                @pl.when(q >= 2)
                def _():
                    nwait_out(q - 2, s)
                ncompute(s)
                nfire_out(q, s)
            return 0

        lax.fori_loop(0, NQ // 2, nchunk, 0)
        nwait_out(NQ - 2, 0)
        nwait_out(NQ - 1, 1)


@jax.jit
def _run(xp):
    mesh = plsc.VectorSubcoreMesh(core_axis_name="c", subcore_axis_name="s")
    f = pl.kernel(
        _body,
        mesh=mesh,
        out_type=jax.ShapeDtypeStruct((4 * 3 * G3,), jnp.float32),
        scratch_types=[
            pltpu.VMEM((PPT,), jnp.float32),
            pltpu.VMEM((PPT,), jnp.float32),
            pltpu.VMEM((PPT,), jnp.float32),
            pltpu.VMEM((128,), jnp.int32),
            pltpu.VMEM((128,), jnp.float32),
            pltpu.VMEM((128,), jnp.float32),
            pltpu.VMEM((128,), jnp.float32),
            pltpu.VMEM((128,), jnp.int32),
            pltpu.VMEM((128,), jnp.float32),
            pltpu.VMEM((128,), jnp.float32),
            pltpu.VMEM((128,), jnp.float32),
            pltpu.VMEM((128,), jnp.float32),
            pltpu.VMEM((QB,), jnp.float32),
            pltpu.VMEM((QB,), jnp.float32),
            pltpu.VMEM((QB,), jnp.float32),
            pltpu.VMEM((QB,), jnp.float32),
            pltpu.VMEM((QB,), jnp.float32),
            pltpu.VMEM((QB,), jnp.float32),
            pltpu.VMEM((QB,), jnp.float32),
            pltpu.VMEM((QB,), jnp.float32),
            pltpu.VMEM((QB,), jnp.float32),
            pltpu.VMEM((QB,), jnp.float32),
            pltpu.VMEM((QB,), jnp.float32),
            pltpu.VMEM((QB,), jnp.float32),
            pltpu.VMEM((QB,), jnp.float32),
            pltpu.VMEM((QB,), jnp.float32),
            pltpu.VMEM((QB,), jnp.float32),
            pltpu.SemaphoreType.DMA,
            pltpu.SemaphoreType.DMA,
            pltpu.SemaphoreType.DMA,
            pltpu.SemaphoreType.DMA,
            pltpu.SemaphoreType.DMA,
            pltpu.SemaphoreType.DMA,
            pltpu.SemaphoreType.DMA,
            pltpu.SemaphoreType.DMA,
            pltpu.VMEM_SHARED((G3 + 16,), jnp.float32),
            pltpu.VMEM_SHARED((G3 + 16,), jnp.float32),
            pltpu.VMEM_SHARED((G3 + 16,), jnp.float32),
            pltpu.VMEM_SHARED((G3 + 16,), jnp.float32),
        ],
    )
    return f(xp)


def kernel(x):
    xp = jnp.pad(x, ((0, 0), (0, 0), (0, NPAD - N)), constant_values=10.0)
    out = _run(xp.reshape(-1))
    return out.reshape(4, 3, GRID, GRID, GRID)


# speedup vs baseline: 3.6719x; 3.6719x over previous
import jax
import jax.numpy as jnp
from jax import lax
from jax.experimental import pallas as pl
from jax.experimental.pallas import tpu as pltpu
from jax.experimental.pallas import tpu_sc as plsc

GRID = 64
G3 = GRID * GRID * GRID
N = 100000
NTILES = 16
PPT = 6272
NPAD = NTILES * PPT
ROWS = PPT // 16
STRIPE = G3 // NTILES
NQ = 8
QB = STRIPE // NQ
THRESH = 0.7569


def _splat(v, dt=jnp.float32):
    return lax.broadcast(jnp.asarray(v, dt), (16,))


def _body(x_hbm, out_hbm, xin0, xin1, xin2, idxb, v0, v1, v2,
          idxc, w0, w1, w2, ones, zbuf,
          na0, na1, na2, na3, no0, no1, no2,
          nb0, nb1, nb2, nb3, np0, np1, np2,
          sema, semb, semz, semx, semn0, semn1, semo0, semo1,
          acc0, acc1, acc2, acc3):
    core = lax.axis_index("c")
    sid = lax.axis_index("s")
    accs = (acc0, acc1, acc2, acc3)
    xins = (xin0, xin1, xin2)
    nas = ((na0, na1, na2, na3), (nb0, nb1, nb2, nb3))
    nos = ((no0, no1, no2), (np0, np1, np2))
    semn = (semn0, semn1)
    semo = (semo0, semo1)

    def zrow(i, _):
        zbuf[pl.ds(i * 16, 16)] = _splat(0.0)
        return 0
    lax.fori_loop(0, QB // 16, zrow, 0)
    for k in range(8):
        ones[pl.ds(k * 16, 16)] = _splat(1.0)
    dumpv = lax.broadcast(G3 + sid * 128, (16,)) + lax.broadcasted_iota(jnp.int32, (16,), 0)

    for rep in range(2):
        b = core * 2 + rep

        base = sid * PPT
        for ax in range(3):
            pltpu.async_copy(
                x_hbm.at[pl.ds((b * 3 + ax) * NPAD + base, PPT)], xins[ax],
                semx)

        def zfire(q):
            for accr in accs:
                pltpu.async_copy(zbuf, accr.at[pl.ds(sid * STRIPE + q * QB, QB)],
                                 semz)

        def zwait(q):
            for accr in accs:
                pltpu.make_async_copy(
                    zbuf, accr.at[pl.ds(sid * STRIPE + q * QB, QB)],
                    semz).wait()

        zfire(0)

        def zplane(q, _):
            zfire(q)
            zwait(q - 1)
            return 0
        lax.fori_loop(1, NQ, zplane, 0)
        zwait(NQ - 1)
        for ax in range(3):
            pltpu.make_async_copy(
                x_hbm.at[pl.ds((b * 3 + ax) * NPAD + base, PPT)], xins[ax],
                semx).wait()
        plsc.subcore_barrier()

        def compute_row(j, idxr, vr):
            p = j * 16
            f32 = jnp.float32
            i32 = jnp.int32
            half = _splat(0.5)
            g = _splat(64.0)
            zi = _splat(0, i32)
            mi = _splat(GRID - 1, i32)
            onei = _splat(1, i32)
            i1 = []
            i2 = []
            r1 = []
            r2 = []
            s1 = []
            s2 = []
            for ax in range(3):
                ta = (xins[ax][pl.ds(p, 16)] + half) * g - half
                ia = ta.astype(i32)
                up = jnp.where(ta > ia.astype(f32), onei, zi)
                ib = ia + up
                ia = jnp.minimum(jnp.maximum(ia, zi), mi)
                ib = jnp.minimum(jnp.maximum(ib, zi), mi)
                ra = ta - ia.astype(f32)
                rb = ta - ib.astype(f32)
                i1.append(ia)
                i2.append(ib)
                r1.append(ra)
                r2.append(rb)
                s1.append(ra * ra)
                s2.append(rb * rb)
            s12 = _splat(12, i32)
            s6 = _splat(6, i32)
            X = (lax.shift_left(i1[0], s12), lax.shift_left(i2[0], s12))
            Y = (lax.shift_left(i1[1], s6), lax.shift_left(i2[1], s6))
            Z = (i1[2], i2[2])
            SX = (s1[0], s2[0])
            SY = (s1[1], s2[1])
            SZ = (s1[2], s2[2])
            RX = (r1[0], r2[0])
            RY = (r1[1], r2[1])
            RZ = (r1[2], r2[2])
            th = _splat(THRESH)
            c = 0
            for a in range(2):
                for bb in range(2):
                    for cc in range(2):
                        d2 = (SX[a] + SY[bb]) + SZ[cc]
                        idx = (X[a] + Y[bb]) + Z[cc]
                        idx = jnp.where(d2 < th, idx, dumpv + _splat(c * 16, i32))
                        sl = pl.ds(c * 16, 16)
                        idxr[sl] = idx
                        vr[0][sl] = RX[a]
                        vr[1][sl] = RY[bb]
                        vr[2][sl] = RZ[cc]
                        c += 1

        def fire(idxr, vr, sem):
            for accr, v in zip(accs, vr + (ones,)):
                pltpu.async_copy(v, accr.at[idxr], sem, add=True)

        def drain(idxr, vr, sem):
            for accr, v in zip(accs, vr + (ones,)):
                pltpu.make_async_copy(v, accr.at[idxr], sem).wait()

        vsa = (v0, v1, v2)
        vsb = (w0, w1, w2)
        compute_row(0, idxb, vsa)
        fire(idxb, vsa, sema)
        compute_row(1, idxc, vsb)
        fire(idxc, vsb, semb)

        def pair(m, _):
            drain(idxb, vsa, sema)
            compute_row(2 * m, idxb, vsa)
            fire(idxb, vsa, sema)
            drain(idxc, vsb, semb)
            compute_row(2 * m + 1, idxc, vsb)
            fire(idxc, vsb, semb)
            return 0

        lax.fori_loop(1, ROWS // 2, pair, 0)
        drain(idxb, vsa, sema)
        drain(idxc, vsb, semb)
        plsc.subcore_barrier()

        def nfire_in(q, s):
            nb = sid * STRIPE + q * QB
            for ch in range(4):
                pltpu.async_copy(accs[ch].at[pl.ds(nb, QB)], nas[s][ch],
                                 semn[s])

        def nwait_in(q, s):
            nb = sid * STRIPE + q * QB
            for ch in range(4):
                pltpu.make_async_copy(accs[ch].at[pl.ds(nb, QB)], nas[s][ch],
                                      semn[s]).wait()

        def nfire_out(q, s):
            nb = sid * STRIPE + q * QB
            for ch in range(3):
                pltpu.async_copy(
                    nos[s][ch], out_hbm.at[pl.ds((b * 3 + ch) * G3 + nb, QB)],
                    semo[s])

        def nwait_out(q, s):
            nb = sid * STRIPE + q * QB
            for ch in range(3):
                pltpu.make_async_copy(
                    nos[s][ch], out_hbm.at[pl.ds((b * 3 + ch) * G3 + nb, QB)],
                    semo[s]).wait()

        def ncompute(s):
            a0, a1, a2, a3 = nas[s]
            o0, o1, o2 = nos[s]

            def nrow(i, _):
                sl = pl.ds(i * 16, 16)
                w = jnp.maximum(a3[sl], _splat(1.0))
                o0[sl] = a0[sl] / w
                o1[sl] = a1[sl] / w
                o2[sl] = a2[sl] / w
                return 0

            lax.fori_loop(0, QB // 16, nrow, 0)

        nfire_in(0, 0)

        def nchunk(h, _):
            for s in range(2):
                q = 2 * h + s
                nwait_in(q, s)

                @pl.when(q + 1 < NQ)
                def _():
                    nfire_in(q + 1, 1 - s)

                @pl.when(q >= 2)
                def _():
                    nwait_out(q - 2, s)
                ncompute(s)
                nfire_out(q, s)
            return 0

        lax.fori_loop(0, NQ // 2, nchunk, 0)
        nwait_out(NQ - 2, 0)
        nwait_out(NQ - 1, 1)


@jax.jit
def _run(xp):
    mesh = plsc.VectorSubcoreMesh(core_axis_name="c", subcore_axis_name="s")
    f = pl.kernel(
        _body,
        mesh=mesh,
        out_type=jax.ShapeDtypeStruct((4 * 3 * G3,), jnp.float32),
        scratch_types=[
            pltpu.VMEM((PPT,), jnp.float32),
            pltpu.VMEM((PPT,), jnp.float32),
            pltpu.VMEM((PPT,), jnp.float32),
            pltpu.VMEM((128,), jnp.int32),
            pltpu.VMEM((128,), jnp.float32),
            pltpu.VMEM((128,), jnp.float32),
            pltpu.VMEM((128,), jnp.float32),
            pltpu.VMEM((128,), jnp.int32),
            pltpu.VMEM((128,), jnp.float32),
            pltpu.VMEM((128,), jnp.float32),
            pltpu.VMEM((128,), jnp.float32),
            pltpu.VMEM((128,), jnp.float32),
            pltpu.VMEM((QB,), jnp.float32),
            pltpu.VMEM((QB,), jnp.float32),
            pltpu.VMEM((QB,), jnp.float32),
            pltpu.VMEM((QB,), jnp.float32),
            pltpu.VMEM((QB,), jnp.float32),
            pltpu.VMEM((QB,), jnp.float32),
            pltpu.VMEM((QB,), jnp.float32),
            pltpu.VMEM((QB,), jnp.float32),
            pltpu.VMEM((QB,), jnp.float32),
            pltpu.VMEM((QB,), jnp.float32),
            pltpu.VMEM((QB,), jnp.float32),
            pltpu.VMEM((QB,), jnp.float32),
            pltpu.VMEM((QB,), jnp.float32),
            pltpu.VMEM((QB,), jnp.float32),
            pltpu.VMEM((QB,), jnp.float32),
            pltpu.SemaphoreType.DMA,
            pltpu.SemaphoreType.DMA,
            pltpu.SemaphoreType.DMA,
            pltpu.SemaphoreType.DMA,
            pltpu.SemaphoreType.DMA,
            pltpu.SemaphoreType.DMA,
            pltpu.SemaphoreType.DMA,
            pltpu.SemaphoreType.DMA,
            pltpu.VMEM_SHARED((G3 + 2048,), jnp.float32),
            pltpu.VMEM_SHARED((G3 + 2048,), jnp.float32),
            pltpu.VMEM_SHARED((G3 + 2048,), jnp.float32),
            pltpu.VMEM_SHARED((G3 + 2048,), jnp.float32),
        ],
    )
    return f(xp)


def kernel(x):
    xp = jnp.pad(x, ((0, 0), (0, 0), (0, NPAD - N)), constant_values=10.0)
    out = _run(xp.reshape(-1))
    return out.reshape(4, 3, GRID, GRID, GRID)


# speedup vs baseline: 3.6768x; 1.0013x over previous
import jax
import jax.numpy as jnp
from jax import lax
from jax.experimental import pallas as pl
from jax.experimental.pallas import tpu as pltpu
from jax.experimental.pallas import tpu_sc as plsc

GRID = 64
G3 = GRID * GRID * GRID
N = 100000
NTILES = 16
PPT = 6272
NPAD = NTILES * PPT
ROWS = PPT // 16
STRIPE = G3 // NTILES
NQ = 8
QB = STRIPE // NQ
THRESH = 0.7569


def _splat(v, dt=jnp.float32):
    return lax.broadcast(jnp.asarray(v, dt), (16,))


def _body(x_hbm, out_hbm, xin0, xin1, xin2, idxb, v0, v1, v2,
          idxc, w0, w1, w2, ones, zbuf,
          na0, na1, na2, na3, no0, no1, no2,
          nb0, nb1, nb2, nb3, np0, np1, np2,
          sema, semb, semz, semx, semn0, semn1, semo0, semo1,
          acc0, acc1, acc2, acc3):
    core = lax.axis_index("c")
    sid = lax.axis_index("s")
    accs = (acc0, acc1, acc2, acc3)
    xins = (xin0, xin1, xin2)
    nas = ((na0, na1, na2, na3), (nb0, nb1, nb2, nb3))
    nos = ((no0, no1, no2), (np0, np1, np2))
    semn = (semn0, semn1)
    semo = (semo0, semo1)

    def zrow(i, _):
        zbuf[pl.ds(i * 16, 16)] = _splat(0.0)
        return 0
    lax.fori_loop(0, QB // 16, zrow, 0)
    for k in range(8):
        ones[pl.ds(k * 16, 16)] = _splat(1.0)
    dumpv = lax.broadcast(G3 + sid * 128, (16,)) + lax.broadcasted_iota(jnp.int32, (16,), 0)

    for rep in range(2):
        b = core * 2 + rep

        base = sid * PPT
        for ax in range(3):
            pltpu.async_copy(
                x_hbm.at[pl.ds((b * 3 + ax) * NPAD + base, PPT)], xins[ax],
                semx)

        def zfire(q):
            for accr in accs:
                pltpu.async_copy(zbuf, accr.at[pl.ds(sid * STRIPE + q * QB, QB)],
                                 semz)

        def zwait(q):
            for accr in accs:
                pltpu.make_async_copy(
                    zbuf, accr.at[pl.ds(sid * STRIPE + q * QB, QB)],
                    semz).wait()

        zfire(0)

        def zplane(q, _):
            zfire(q)
            zwait(q - 1)
            return 0
        lax.fori_loop(1, NQ, zplane, 0)
        zwait(NQ - 1)
        for ax in range(3):
            pltpu.make_async_copy(
                x_hbm.at[pl.ds((b * 3 + ax) * NPAD + base, PPT)], xins[ax],
                semx).wait()
        plsc.subcore_barrier()

        def compute_row(j, idxr, vr):
            p = j * 16
            f32 = jnp.float32
            i32 = jnp.int32
            half = _splat(0.5)
            g = _splat(64.0)
            zi = _splat(0, i32)
            mi = _splat(GRID - 1, i32)
            onei = _splat(1, i32)
            i1 = []
            i2 = []
            r1 = []
            r2 = []
            s1 = []
            s2 = []
            for ax in range(3):
                ta = (xins[ax][pl.ds(p, 16)] + half) * g - half
                ia = ta.astype(i32)
                up = jnp.where(ta > ia.astype(f32), onei, zi)
                ib = ia + up
                ia = jnp.minimum(jnp.maximum(ia, zi), mi)
                ib = jnp.minimum(jnp.maximum(ib, zi), mi)
                ra = ta - ia.astype(f32)
                rb = ta - ib.astype(f32)
                i1.append(ia)
                i2.append(ib)
                r1.append(ra)
                r2.append(rb)
                s1.append(ra * ra)
                s2.append(rb * rb)
            s12 = _splat(12, i32)
            s6 = _splat(6, i32)
            X = (lax.shift_left(i1[0], s12), lax.shift_left(i2[0], s12))
            Y = (lax.shift_left(i1[1], s6), lax.shift_left(i2[1], s6))
            Z = (i1[2], i2[2])
            SX = (s1[0], s2[0])
            SY = (s1[1], s2[1])
            SZ = (s1[2], s2[2])
            RX = (r1[0], r2[0])
            RY = (r1[1], r2[1])
            RZ = (r1[2], r2[2])
            th = _splat(THRESH)
            c = 0
            for a in range(2):
                for bb in range(2):
                    for cc in range(2):
                        d2 = (SX[a] + SY[bb]) + SZ[cc]
                        idx = (X[a] + Y[bb]) + Z[cc]
                        idx = jnp.where(d2 < th, idx, dumpv + _splat(c * 16, i32))
                        sl = pl.ds(c * 16, 16)
                        idxr[sl] = idx
                        vr[0][sl] = RX[a]
                        vr[1][sl] = RY[bb]
                        vr[2][sl] = RZ[cc]
                        c += 1

        def fire(idxr, vr, sem):
            for accr, v in zip(accs, vr + (ones,)):
                pltpu.async_copy(v, accr.at[idxr], sem, add=True)

        def drain(idxr, vr, sem):
            for accr, v in zip(accs, vr + (ones,)):
                pltpu.make_async_copy(v, accr.at[idxr], sem).wait()

        vsa = (v0, v1, v2)
        vsb = (w0, w1, w2)
        compute_row(0, idxb, vsa)
        fire(idxb, vsa, sema)
        compute_row(1, idxc, vsb)
        fire(idxc, vsb, semb)

        def pair(m, _):
            drain(idxb, vsa, sema)
            compute_row(2 * m, idxb, vsa)
            fire(idxb, vsa, sema)
            drain(idxc, vsb, semb)
            compute_row(2 * m + 1, idxc, vsb)
            fire(idxc, vsb, semb)
            return 0

        lax.fori_loop(1, ROWS // 2, pair, 0)
        drain(idxb, vsa, sema)
        drain(idxc, vsb, semb)
        plsc.subcore_barrier()

        def nfire_in(q, s):
            nb = sid * STRIPE + q * QB
            for ch in range(4):
                pltpu.async_copy(accs[ch].at[pl.ds(nb, QB)], nas[s][ch],
                                 semn[s])

        def nwait_in(q, s):
            nb = sid * STRIPE + q * QB
            for ch in range(4):
                pltpu.make_async_copy(accs[ch].at[pl.ds(nb, QB)], nas[s][ch],
                                      semn[s]).wait()

        def nfire_out(q, s):
            nb = sid * STRIPE + q * QB
            for ch in range(3):
                pltpu.async_copy(
                    nos[s][ch], out_hbm.at[pl.ds((b * 3 + ch) * G3 + nb, QB)],
                    semo[s])

        def nwait_out(q, s):
            nb = sid * STRIPE + q * QB
            for ch in range(3):
                pltpu.make_async_copy(
                    nos[s][ch], out_hbm.at[pl.ds((b * 3 + ch) * G3 + nb, QB)],
                    semo[s]).wait()

        def ncompute(s):
            a0, a1, a2, a3 = nas[s]
            o0, o1, o2 = nos[s]

            def nrow(i, _):
                sl = pl.ds(i * 16, 16)
                rw = _splat(1.0) / jnp.maximum(a3[sl], _splat(1.0))
                o0[sl] = a0[sl] * rw
                o1[sl] = a1[sl] * rw
                o2[sl] = a2[sl] * rw
                return 0

            lax.fori_loop(0, QB // 16, nrow, 0)

        nfire_in(0, 0)

        def nchunk(h, _):
            for s in range(2):
                q = 2 * h + s
                nwait_in(q, s)

                @pl.when(q + 1 < NQ)
                def _():
                    nfire_in(q + 1, 1 - s)

                @pl.when(q >= 2)
                def _():
                    nwait_out(q - 2, s)
                ncompute(s)
                nfire_out(q, s)
            return 0

        lax.fori_loop(0, NQ // 2, nchunk, 0)
        nwait_out(NQ - 2, 0)
        nwait_out(NQ - 1, 1)


@jax.jit
def _run(xp):
    mesh = plsc.VectorSubcoreMesh(core_axis_name="c", subcore_axis_name="s")
    f = pl.kernel(
        _body,
        mesh=mesh,
        out_type=jax.ShapeDtypeStruct((4 * 3 * G3,), jnp.float32),
        scratch_types=[
            pltpu.VMEM((PPT,), jnp.float32),
            pltpu.VMEM((PPT,), jnp.float32),
            pltpu.VMEM((PPT,), jnp.float32),
            pltpu.VMEM((128,), jnp.int32),
            pltpu.VMEM((128,), jnp.float32),
            pltpu.VMEM((128,), jnp.float32),
            pltpu.VMEM((128,), jnp.float32),
            pltpu.VMEM((128,), jnp.int32),
            pltpu.VMEM((128,), jnp.float32),
            pltpu.VMEM((128,), jnp.float32),
            pltpu.VMEM((128,), jnp.float32),
            pltpu.VMEM((128,), jnp.float32),
            pltpu.VMEM((QB,), jnp.float32),
            pltpu.VMEM((QB,), jnp.float32),
            pltpu.VMEM((QB,), jnp.float32),
            pltpu.VMEM((QB,), jnp.float32),
            pltpu.VMEM((QB,), jnp.float32),
            pltpu.VMEM((QB,), jnp.float32),
            pltpu.VMEM((QB,), jnp.float32),
            pltpu.VMEM((QB,), jnp.float32),
            pltpu.VMEM((QB,), jnp.float32),
            pltpu.VMEM((QB,), jnp.float32),
            pltpu.VMEM((QB,), jnp.float32),
            pltpu.VMEM((QB,), jnp.float32),
            pltpu.VMEM((QB,), jnp.float32),
            pltpu.VMEM((QB,), jnp.float32),
            pltpu.VMEM((QB,), jnp.float32),
            pltpu.SemaphoreType.DMA,
            pltpu.SemaphoreType.DMA,
            pltpu.SemaphoreType.DMA,
            pltpu.SemaphoreType.DMA,
            pltpu.SemaphoreType.DMA,
            pltpu.SemaphoreType.DMA,
            pltpu.SemaphoreType.DMA,
            pltpu.SemaphoreType.DMA,
            pltpu.VMEM_SHARED((G3 + 2048,), jnp.float32),
            pltpu.VMEM_SHARED((G3 + 2048,), jnp.float32),
            pltpu.VMEM_SHARED((G3 + 2048,), jnp.float32),
            pltpu.VMEM_SHARED((G3 + 2048,), jnp.float32),
        ],
    )
    return f(xp)


def kernel(x):
    xp = jnp.pad(x, ((0, 0), (0, 0), (0, NPAD - N)), constant_values=10.0)
    out = _run(xp.reshape(-1))
    return out.reshape(4, 3, GRID, GRID, GRID)


# speedup vs baseline: 3.6911x; 1.0039x over previous
import jax
import jax.numpy as jnp
from jax import lax
from jax.experimental import pallas as pl
from jax.experimental.pallas import tpu as pltpu
from jax.experimental.pallas import tpu_sc as plsc

GRID = 64
G3 = GRID * GRID * GRID
N = 100000
NTILES = 16
PPT = 6272
NPAD = NTILES * PPT
ROWS = PPT // 16
STRIPE = G3 // NTILES
NQ = 8
QB = STRIPE // NQ
THRESH = 0.7569


def _splat(v, dt=jnp.float32):
    return lax.broadcast(jnp.asarray(v, dt), (16,))


def _body(x_hbm, out_hbm, xin0, xin1, xin2, idxb, v0, v1, v2,
          idxc, w0, w1, w2, ones, zbuf,
          na0, na1, na2, na3, no0, no1, no2,
          nb0, nb1, nb2, nb3, np0, np1, np2,
          sema, semb, semz, semx, semn0, semn1, semo0, semo1,
          acc0, acc1, acc2, acc3):
    core = lax.axis_index("c")
    sid = lax.axis_index("s")
    accs = (acc0, acc1, acc2, acc3)
    xins = (xin0, xin1, xin2)
    nas = ((na0, na1, na2, na3), (nb0, nb1, nb2, nb3))
    nos = ((no0, no1, no2), (np0, np1, np2))
    semn = (semn0, semn1)
    semo = (semo0, semo1)

    def zrow(i, _):
        zbuf[pl.ds(i * 16, 16)] = _splat(0.0)
        return 0
    lax.fori_loop(0, QB // 16, zrow, 0)
    for k in range(8):
        ones[pl.ds(k * 16, 16)] = _splat(1.0)
    dumpv = lax.broadcast(G3 + sid * 128, (16,)) + lax.broadcasted_iota(jnp.int32, (16,), 0)

    def zfire(q):
        for accr in accs:
            pltpu.async_copy(zbuf, accr.at[pl.ds(sid * STRIPE + q * QB, QB)],
                             semz)

    def zwait(q):
        for accr in accs:
            pltpu.make_async_copy(
                zbuf, accr.at[pl.ds(sid * STRIPE + q * QB, QB)], semz).wait()

    for rep in range(2):
        b = core * 2 + rep

        if rep == 0:
            base = sid * PPT
            for ax in range(3):
                pltpu.async_copy(
                    x_hbm.at[pl.ds((b * 3 + ax) * NPAD + base, PPT)],
                    xins[ax], semx)
            zfire(0)

            def zplane(q, _):
                zfire(q)
                zwait(q - 1)
                return 0
            lax.fori_loop(1, NQ, zplane, 0)
            zwait(NQ - 1)
            for ax in range(3):
                pltpu.make_async_copy(
                    x_hbm.at[pl.ds((b * 3 + ax) * NPAD + base, PPT)],
                    xins[ax], semx).wait()
        else:
            base = sid * PPT
            for ax in range(3):
                pltpu.make_async_copy(
                    x_hbm.at[pl.ds((b * 3 + ax) * NPAD + base, PPT)],
                    xins[ax], semx).wait()
            zwait(NQ - 1)
        plsc.subcore_barrier()

        def compute_row(j, idxr, vr):
            p = j * 16
            f32 = jnp.float32
            i32 = jnp.int32
            half = _splat(0.5)
            g = _splat(64.0)
            zi = _splat(0, i32)
            mi = _splat(GRID - 1, i32)
            onei = _splat(1, i32)
            i1 = []
            i2 = []
            r1 = []
            r2 = []
            s1 = []
            s2 = []
            for ax in range(3):
                ta = (xins[ax][pl.ds(p, 16)] + half) * g - half
                ia = ta.astype(i32)
                up = jnp.where(ta > ia.astype(f32), onei, zi)
                ib = ia + up
                ia = jnp.minimum(jnp.maximum(ia, zi), mi)
                ib = jnp.minimum(jnp.maximum(ib, zi), mi)
                ra = ta - ia.astype(f32)
                rb = ta - ib.astype(f32)
                i1.append(ia)
                i2.append(ib)
                r1.append(ra)
                r2.append(rb)
                s1.append(ra * ra)
                s2.append(rb * rb)
            s12 = _splat(12, i32)
            s6 = _splat(6, i32)
            X = (lax.shift_left(i1[0], s12), lax.shift_left(i2[0], s12))
            Y = (lax.shift_left(i1[1], s6), lax.shift_left(i2[1], s6))
            Z = (i1[2], i2[2])
            SX = (s1[0], s2[0])
            SY = (s1[1], s2[1])
            SZ = (s1[2], s2[2])
            RX = (r1[0], r2[0])
            RY = (r1[1], r2[1])
            RZ = (r1[2], r2[2])
            th = _splat(THRESH)
            c = 0
            for a in range(2):
                for bb in range(2):
                    for cc in range(2):
                        d2 = (SX[a] + SY[bb]) + SZ[cc]
                        idx = (X[a] + Y[bb]) + Z[cc]
                        idx = jnp.where(d2 < th, idx, dumpv + _splat(c * 16, i32))
                        sl = pl.ds(c * 16, 16)
                        idxr[sl] = idx
                        vr[0][sl] = RX[a]
                        vr[1][sl] = RY[bb]
                        vr[2][sl] = RZ[cc]
                        c += 1

        def fire(idxr, vr, sem):
            for accr, v in zip(accs, vr + (ones,)):
                pltpu.async_copy(v, accr.at[idxr], sem, add=True)

        def drain(idxr, vr, sem):
            for accr, v in zip(accs, vr + (ones,)):
                pltpu.make_async_copy(v, accr.at[idxr], sem).wait()

        vsa = (v0, v1, v2)
        vsb = (w0, w1, w2)
        compute_row(0, idxb, vsa)
        fire(idxb, vsa, sema)
        compute_row(1, idxc, vsb)
        fire(idxc, vsb, semb)

        def pair(m, _):
            drain(idxb, vsa, sema)
            compute_row(2 * m, idxb, vsa)
            fire(idxb, vsa, sema)
            drain(idxc, vsb, semb)
            compute_row(2 * m + 1, idxc, vsb)
            fire(idxc, vsb, semb)
            return 0

        lax.fori_loop(1, ROWS // 2, pair, 0)
        drain(idxb, vsa, sema)
        drain(idxc, vsb, semb)
        plsc.subcore_barrier()
        if rep == 0:
            nbase = sid * PPT
            for ax in range(3):
                pltpu.async_copy(
                    x_hbm.at[pl.ds(((b + 1) * 3 + ax) * NPAD + nbase, PPT)],
                    xins[ax], semx)

        def nfire_in(q, s):
            nb = sid * STRIPE + q * QB
            for ch in range(4):
                pltpu.async_copy(accs[ch].at[pl.ds(nb, QB)], nas[s][ch],
                                 semn[s])

        def nwait_in(q, s):
            nb = sid * STRIPE + q * QB
            for ch in range(4):
                pltpu.make_async_copy(accs[ch].at[pl.ds(nb, QB)], nas[s][ch],
                                      semn[s]).wait()

        def nfire_out(q, s):
            nb = sid * STRIPE + q * QB
            for ch in range(3):
                pltpu.async_copy(
                    nos[s][ch], out_hbm.at[pl.ds((b * 3 + ch) * G3 + nb, QB)],
                    semo[s])

        def nwait_out(q, s):
            nb = sid * STRIPE + q * QB
            for ch in range(3):
                pltpu.make_async_copy(
                    nos[s][ch], out_hbm.at[pl.ds((b * 3 + ch) * G3 + nb, QB)],
                    semo[s]).wait()

        def ncompute(s):
            a0, a1, a2, a3 = nas[s]
            o0, o1, o2 = nos[s]

            def nrow(i, _):
                sl = pl.ds(i * 16, 16)
                w = jnp.maximum(a3[sl], _splat(1.0))
                o0[sl] = a0[sl] / w
                o1[sl] = a1[sl] / w
                o2[sl] = a2[sl] / w
                return 0

            lax.fori_loop(0, QB // 16, nrow, 0)

        nfire_in(0, 0)

        def nchunk(h, _):
            for s in range(2):
                q = 2 * h + s
                nwait_in(q, s)

                @pl.when(q + 1 < NQ)
                def _():
                    nfire_in(q + 1, 1 - s)

                @pl.when(q >= 2)
                def _():
                    nwait_out(q - 2, s)
                ncompute(s)
                nfire_out(q, s)
                if rep == 0:
                    @pl.when(q >= 1)
                    def _():
                        zwait(q - 1)
                    zfire(q)
            return 0

        lax.fori_loop(0, NQ // 2, nchunk, 0)
        nwait_out(NQ - 2, 0)
        nwait_out(NQ - 1, 1)


@jax.jit
def _run(xp):
    mesh = plsc.VectorSubcoreMesh(core_axis_name="c", subcore_axis_name="s")
    f = pl.kernel(
        _body,
        mesh=mesh,
        out_type=jax.ShapeDtypeStruct((4 * 3 * G3,), jnp.float32),
        scratch_types=[
            pltpu.VMEM((PPT,), jnp.float32),
            pltpu.VMEM((PPT,), jnp.float32),
            pltpu.VMEM((PPT,), jnp.float32),
            pltpu.VMEM((128,), jnp.int32),
            pltpu.VMEM((128,), jnp.float32),
            pltpu.VMEM((128,), jnp.float32),
            pltpu.VMEM((128,), jnp.float32),
            pltpu.VMEM((128,), jnp.int32),
            pltpu.VMEM((128,), jnp.float32),
            pltpu.VMEM((128,), jnp.float32),
            pltpu.VMEM((128,), jnp.float32),
            pltpu.VMEM((128,), jnp.float32),
            pltpu.VMEM((QB,), jnp.float32),
            pltpu.VMEM((QB,), jnp.float32),
            pltpu.VMEM((QB,), jnp.float32),
            pltpu.VMEM((QB,), jnp.float32),
            pltpu.VMEM((QB,), jnp.float32),
            pltpu.VMEM((QB,), jnp.float32),
            pltpu.VMEM((QB,), jnp.float32),
            pltpu.VMEM((QB,), jnp.float32),
            pltpu.VMEM((QB,), jnp.float32),
            pltpu.VMEM((QB,), jnp.float32),
            pltpu.VMEM((QB,), jnp.float32),
            pltpu.VMEM((QB,), jnp.float32),
            pltpu.VMEM((QB,), jnp.float32),
            pltpu.VMEM((QB,), jnp.float32),
            pltpu.VMEM((QB,), jnp.float32),
            pltpu.SemaphoreType.DMA,
            pltpu.SemaphoreType.DMA,
            pltpu.SemaphoreType.DMA,
            pltpu.SemaphoreType.DMA,
            pltpu.SemaphoreType.DMA,
            pltpu.SemaphoreType.DMA,
            pltpu.SemaphoreType.DMA,
            pltpu.SemaphoreType.DMA,
            pltpu.VMEM_SHARED((G3 + 2048,), jnp.float32),
            pltpu.VMEM_SHARED((G3 + 2048,), jnp.float32),
            pltpu.VMEM_SHARED((G3 + 2048,), jnp.float32),
            pltpu.VMEM_SHARED((G3 + 2048,), jnp.float32),
        ],
    )
    return f(xp)


def kernel(x):
    xp = jnp.pad(x, ((0, 0), (0, 0), (0, NPAD - N)), constant_values=10.0)
    out = _run(xp.reshape(-1))
    return out.reshape(4, 3, GRID, GRID, GRID)
